# baseline (speedup 1.0000x reference)
import functools

import jax
import jax.numpy as jnp
from jax import lax
from jax.experimental import pallas as pl
from jax.experimental.pallas import tpu as pltpu
from jax.experimental.pallas import tpu_sc as plsc

B = 8
N_FINE = 40962
D = 128
K = 7
N_COARSE = 10242

NW = 32
PER_W = 320
N_MAIN = NW * PER_W
N_TAIL = N_COARSE - N_MAIN
CHUNK = 16
NCHUNK = PER_W // CHUNK
IDX_C = CHUNK * K
DBLK = D // 16


def _pool_body(x_hbm, idxm_hbm, idxt_hbm, out_hbm,
               idx_v, idx_adj, gbuf, obuf, idxt_raw, idxt_v, gtail, sem):
    wid = lax.axis_index("s") * 2 + lax.axis_index("c")
    base_c = wid * PER_W
    pltpu.sync_copy(idxm_hbm.at[pl.ds(wid * (PER_W * K), PER_W * K)], idx_v)

    def batch_body(b, carry):
        off = b * N_FINE

        def chunk_body(j, carry2):
            for t in range(IDX_C // 16):
                idx_adj[pl.ds(t * 16, 16)] = idx_v[pl.ds(j * IDX_C + t * 16, 16)] + off
            pltpu.async_copy(x_hbm.at[idx_adj], gbuf, sem).wait()

            def cbody(c, carry3):
                for dblk in range(DBLK):
                    o = pl.ds(dblk * 16, 16)
                    m = gbuf[K * c, o]
                    for k2 in range(1, K):
                        m = jnp.maximum(m, gbuf[K * c + k2, o])
                    obuf[c, o] = m
                return carry3

            lax.fori_loop(0, CHUNK, cbody, 0)
            pltpu.sync_copy(obuf, out_hbm.at[b, pl.ds(base_c + j * CHUNK, CHUNK)])
            return carry2

        lax.fori_loop(0, NCHUNK, chunk_body, 0)
        return carry

    lax.fori_loop(0, B, batch_body, 0)

    @pl.when(wid == NW - 1)
    def _():
        pltpu.sync_copy(idxt_hbm, idxt_raw)
        for b in range(B):
            off = b * N_FINE
            idxt_v[pl.ds(0, 16)] = idxt_raw[pl.ds(0, 16)] + off
            pltpu.async_copy(x_hbm.at[idxt_v], gtail, sem).wait()
            for c in range(N_TAIL):
                for dblk in range(DBLK):
                    o = pl.ds(dblk * 16, 16)
                    m = gtail[K * c, o]
                    for k2 in range(1, K):
                        m = jnp.maximum(m, gtail[K * c + k2, o])
                    obuf[c, o] = m
            pltpu.sync_copy(obuf.at[pl.ds(0, N_TAIL)],
                            out_hbm.at[b, pl.ds(N_MAIN, N_TAIL)])


def kernel(x, pool_idx):
    idx = pool_idx.astype(jnp.int32)
    idx_main = idx[:N_MAIN].reshape(N_MAIN * K)
    idx_tail = jnp.pad(idx[N_MAIN:].reshape(N_TAIL * K), (0, 16 - N_TAIL * K))
    x_flat = x.reshape(B * N_FINE, D)

    mesh = plsc.VectorSubcoreMesh(core_axis_name="c", subcore_axis_name="s")
    f = pl.kernel(
        _pool_body,
        mesh=mesh,
        out_type=jax.ShapeDtypeStruct((B, N_COARSE, D), jnp.float32),
        scratch_types=[
            pltpu.VMEM((PER_W * K,), jnp.int32),
            pltpu.VMEM((IDX_C,), jnp.int32),
            pltpu.VMEM((IDX_C, D), jnp.float32),
            pltpu.VMEM((CHUNK, D), jnp.float32),
            pltpu.VMEM((16,), jnp.int32),
            pltpu.VMEM((16,), jnp.int32),
            pltpu.VMEM((16, D), jnp.float32),
            pltpu.SemaphoreType.DMA,
        ],
    )
    return f(x_flat, idx_main, idx_tail)


# speedup vs baseline: 1.3520x; 1.3520x over previous
import jax
import jax.numpy as jnp
from jax import lax
from jax.experimental import pallas as pl
from jax.experimental.pallas import tpu as pltpu
from jax.experimental.pallas import tpu_sc as plsc

B = 8
N_FINE = 40962
D = 128
K = 7
N_COARSE = 10242

NW = 32
PER_W = 320
N_MAIN = NW * PER_W
N_TAIL = N_COARSE - N_MAIN
CHUNK = 16
NCHUNK = PER_W // CHUNK
IDX_C = CHUNK * K
DBLK = D // 16


def _pool_body(x_hbm, idxm_hbm, idxt_hbm, out_hbm,
               idx_v, idx_adjA, idx_adjB, gbufA, gbufB, obuf,
               idxt_raw, idxt_adj, gtail, otail,
               semA, semB, semS, semT):
    wid = lax.axis_index("s") * 2 + lax.axis_index("c")
    base_c = wid * PER_W
    pltpu.sync_copy(idxm_hbm.at[pl.ds(wid * (PER_W * K), PER_W * K)], idx_v)

    def fire(idx_adj, gbuf, sem, j_words, off):
        for t in range(IDX_C // 16):
            idx_adj[pl.ds(t * 16, 16)] = idx_v[pl.ds(j_words + t * 16, 16)] + off
        pltpu.async_copy(x_hbm.at[idx_adj], gbuf, sem)

    def wait_gather(idx_adj, gbuf, sem):
        pltpu.make_async_copy(x_hbm.at[idx_adj], gbuf, sem).wait()

    def compute(gbuf, row_base):
        def cbody(c, carry):
            for dblk in range(DBLK):
                o = pl.ds(dblk * 16, 16)
                m = gbuf[K * c, o]
                for k2 in range(1, K):
                    m = jnp.maximum(m, gbuf[K * c + k2, o])
                obuf[row_base + c, o] = m
            return carry
        lax.fori_loop(0, CHUNK, cbody, 0)

    fire(idx_adjA, gbufA, semA, 0, 0)
    fire(idx_adjB, gbufB, semB, IDX_C, 0)

    store_handle = None
    for b in range(B):
        off = b * N_FINE
        if store_handle is not None:
            store_handle.wait()

        def pair_body(j2, carry):
            c0w = (2 * j2) * IDX_C
            wait_gather(idx_adjA, gbufA, semA)
            compute(gbufA, (2 * j2) * CHUNK)
            fire(idx_adjA, gbufA, semA, c0w + 2 * IDX_C, off)
            wait_gather(idx_adjB, gbufB, semB)
            compute(gbufB, (2 * j2 + 1) * CHUNK)
            fire(idx_adjB, gbufB, semB, c0w + 3 * IDX_C, off)
            return carry

        lax.fori_loop(0, NCHUNK // 2 - 1, pair_body, 0)

        wait_gather(idx_adjA, gbufA, semA)
        compute(gbufA, (NCHUNK - 2) * CHUNK)
        if b + 1 < B:
            fire(idx_adjA, gbufA, semA, 0, (b + 1) * N_FINE)
        wait_gather(idx_adjB, gbufB, semB)
        compute(gbufB, (NCHUNK - 1) * CHUNK)
        if b + 1 < B:
            fire(idx_adjB, gbufB, semB, IDX_C, (b + 1) * N_FINE)
        store_handle = pltpu.async_copy(
            obuf, out_hbm.at[b, pl.ds(base_c, PER_W)], semS)

    store_handle.wait()

    @pl.when(wid == NW - 1)
    def _():
        pltpu.sync_copy(idxt_hbm, idxt_raw)

        def tfire(b, carry):
            idxt_adj[b, pl.ds(0, 16)] = idxt_raw[pl.ds(0, 16)] + b * N_FINE
            pltpu.async_copy(x_hbm.at[idxt_adj.at[b]], gtail.at[b], semT)
            return carry

        lax.fori_loop(0, B, tfire, 0)

        def tdrain(b, carry):
            pltpu.make_async_copy(x_hbm.at[idxt_adj.at[b]], gtail.at[b],
                                  semT).wait()
            for c in range(N_TAIL):
                for dblk in range(DBLK):
                    o = pl.ds(dblk * 16, 16)
                    m = gtail[b, K * c, o]
                    for k2 in range(1, K):
                        m = jnp.maximum(m, gtail[b, K * c + k2, o])
                    otail[c, o] = m
            pltpu.sync_copy(otail, out_hbm.at[b, pl.ds(N_MAIN, N_TAIL)])
            return carry

        lax.fori_loop(0, B, tdrain, 0)


def kernel(x, pool_idx):
    idx = pool_idx.astype(jnp.int32)
    idx_main = idx[:N_MAIN].reshape(N_MAIN * K)
    idx_tail = jnp.pad(idx[N_MAIN:].reshape(N_TAIL * K), (0, 16 - N_TAIL * K))
    x_flat = x.reshape(B * N_FINE, D)

    mesh = plsc.VectorSubcoreMesh(core_axis_name="c", subcore_axis_name="s")
    f = pl.kernel(
        _pool_body,
        mesh=mesh,
        out_type=jax.ShapeDtypeStruct((B, N_COARSE, D), jnp.float32),
        scratch_types=[
            pltpu.VMEM((PER_W * K,), jnp.int32),
            pltpu.VMEM((IDX_C,), jnp.int32),
            pltpu.VMEM((IDX_C,), jnp.int32),
            pltpu.VMEM((IDX_C, D), jnp.float32),
            pltpu.VMEM((IDX_C, D), jnp.float32),
            pltpu.VMEM((PER_W, D), jnp.float32),
            pltpu.VMEM((16,), jnp.int32),
            pltpu.VMEM((B, 16), jnp.int32),
            pltpu.VMEM((B, 16, D), jnp.float32),
            pltpu.VMEM((N_TAIL, D), jnp.float32),
            pltpu.SemaphoreType.DMA,
            pltpu.SemaphoreType.DMA,
            pltpu.SemaphoreType.DMA,
            pltpu.SemaphoreType.DMA,
        ],
    )
    return f(x_flat, idx_main, idx_tail)


# speedup vs baseline: 1.7933x; 1.3264x over previous
import jax
import jax.numpy as jnp
from jax import lax
from jax.experimental import pallas as pl
from jax.experimental.pallas import tpu as pltpu
from jax.experimental.pallas import tpu_sc as plsc

B = 8
N_FINE = 40962
D = 128
K = 7
N_COARSE = 10242

NW = 32
PER_W = 320
N_MAIN = NW * PER_W
N_TAIL = N_COARSE - N_MAIN
CHUNK = 16
NCHUNK = PER_W // CHUNK
IDX_C = CHUNK * K
DBLK = D // 16


def _tree_max7(rows):
    t0 = jnp.maximum(rows[0], rows[1])
    t1 = jnp.maximum(rows[2], rows[3])
    t2 = jnp.maximum(rows[4], rows[5])
    return jnp.maximum(jnp.maximum(t0, t1), jnp.maximum(t2, rows[6]))


def _pool_body(x_hbm, idxm_hbm, idxt_hbm, out_hbm,
               idx_v, gbufA, gbufB, obuf,
               idxt_raw, gtail, otail,
               semA, semB, semS, semT):
    wid = lax.axis_index("s") * 2 + lax.axis_index("c")
    base_c = wid * PER_W
    pltpu.sync_copy(idxm_hbm.at[pl.ds(wid * (PER_W * K), PER_W * K)], idx_v)

    def fire(gbuf, sem, j, b):
        idx_slice = idx_v.at[pl.ds(j * IDX_C, IDX_C)]
        pltpu.async_copy(x_hbm.at[b].at[idx_slice], gbuf, sem)

    def wait_gather(gbuf, sem, b):
        idx_slice = idx_v.at[pl.ds(0, IDX_C)]
        pltpu.make_async_copy(x_hbm.at[b].at[idx_slice], gbuf, sem).wait()

    def compute(gbuf, row_base):
        def cbody(c, carry):
            for dblk in range(DBLK):
                o = pl.ds(dblk * 16, 16)
                m = _tree_max7([gbuf[K * c + k2, o] for k2 in range(K)])
                obuf[row_base + c, o] = m
            return carry
        lax.fori_loop(0, CHUNK, cbody, 0)

    fire(gbufA, semA, 0, 0)
    fire(gbufB, semB, 1, 0)

    store_handle = None
    for b in range(B):
        if store_handle is not None:
            store_handle.wait()

        def pair_body(j2, carry):
            c0 = 2 * j2
            wait_gather(gbufA, semA, b)
            compute(gbufA, c0 * CHUNK)
            fire(gbufA, semA, c0 + 2, b)
            wait_gather(gbufB, semB, b)
            compute(gbufB, (c0 + 1) * CHUNK)
            fire(gbufB, semB, c0 + 3, b)
            return carry

        lax.fori_loop(0, NCHUNK // 2 - 1, pair_body, 0)

        wait_gather(gbufA, semA, b)
        compute(gbufA, (NCHUNK - 2) * CHUNK)
        if b + 1 < B:
            fire(gbufA, semA, 0, b + 1)
        wait_gather(gbufB, semB, b)
        compute(gbufB, (NCHUNK - 1) * CHUNK)
        if b + 1 < B:
            fire(gbufB, semB, 1, b + 1)
        store_handle = pltpu.async_copy(
            obuf, out_hbm.at[b, pl.ds(base_c, PER_W)], semS)

    store_handle.wait()

    @pl.when(wid == NW - 1)
    def _():
        pltpu.sync_copy(idxt_hbm, idxt_raw)

        def tfire(b, carry):
            pltpu.async_copy(x_hbm.at[b].at[idxt_raw], gtail.at[b], semT)
            return carry

        lax.fori_loop(0, B, tfire, 0)

        def tdrain(b, carry):
            pltpu.make_async_copy(x_hbm.at[b].at[idxt_raw], gtail.at[b],
                                  semT).wait()
            for c in range(N_TAIL):
                for dblk in range(DBLK):
                    o = pl.ds(dblk * 16, 16)
                    m = _tree_max7([gtail[b, K * c + k2, o]
                                    for k2 in range(K)])
                    otail[c, o] = m
            pltpu.sync_copy(otail, out_hbm.at[b, pl.ds(N_MAIN, N_TAIL)])
            return carry

        lax.fori_loop(0, B, tdrain, 0)


def kernel(x, pool_idx):
    idx = pool_idx.astype(jnp.int32)
    idx_main = idx[:N_MAIN].reshape(N_MAIN * K)
    idx_tail = jnp.pad(idx[N_MAIN:].reshape(N_TAIL * K), (0, 16 - N_TAIL * K))

    mesh = plsc.VectorSubcoreMesh(core_axis_name="c", subcore_axis_name="s")
    f = pl.kernel(
        _pool_body,
        mesh=mesh,
        out_type=jax.ShapeDtypeStruct((B, N_COARSE, D), jnp.float32),
        scratch_types=[
            pltpu.VMEM((PER_W * K,), jnp.int32),
            pltpu.VMEM((IDX_C, D), jnp.float32),
            pltpu.VMEM((IDX_C, D), jnp.float32),
            pltpu.VMEM((PER_W, D), jnp.float32),
            pltpu.VMEM((16,), jnp.int32),
            pltpu.VMEM((B, 16, D), jnp.float32),
            pltpu.VMEM((N_TAIL, D), jnp.float32),
            pltpu.SemaphoreType.DMA,
            pltpu.SemaphoreType.DMA,
            pltpu.SemaphoreType.DMA,
            pltpu.SemaphoreType.DMA,
        ],
    )
    return f(x, idx_main, idx_tail)


# speedup vs baseline: 2.0537x; 1.1452x over previous
import jax
import jax.numpy as jnp
from jax import lax
from jax.experimental import pallas as pl
from jax.experimental.pallas import tpu as pltpu
from jax.experimental.pallas import tpu_sc as plsc

B = 8
N_FINE = 40962
D = 128
K = 7
N_COARSE = 10242

NW = 32
PER_W = 320
N_MAIN = NW * PER_W
N_TAIL = N_COARSE - N_MAIN
CHUNK = 16
NCHUNK = PER_W // CHUNK
IDX_C = CHUNK * K
DBLK = D // 16


def _tree_max7(rows):
    t0 = jnp.maximum(rows[0], rows[1])
    t1 = jnp.maximum(rows[2], rows[3])
    t2 = jnp.maximum(rows[4], rows[5])
    return jnp.maximum(jnp.maximum(t0, t1), jnp.maximum(t2, rows[6]))


def _pool_body(x_hbm, idxm_hbm, idxt_hbm, out_hbm,
               idx_v, gbufA, gbufB, obuf,
               idxt_raw, gtail, otail,
               semA, semB, semS, semT):
    wid = lax.axis_index("s") * 2 + lax.axis_index("c")
    base_c = wid * PER_W
    pltpu.sync_copy(idxm_hbm.at[pl.ds(wid * (PER_W * K), PER_W * K)], idx_v)

    def fire(gbuf, sem, j, b):
        idx_slice = idx_v.at[pl.ds(j * IDX_C, IDX_C)]
        pltpu.async_copy(x_hbm.at[b].at[idx_slice], gbuf, sem)

    def wait_gather(gbuf, sem):
        idx_slice = idx_v.at[pl.ds(0, IDX_C)]
        pltpu.make_async_copy(x_hbm.at[0].at[idx_slice], gbuf, sem).wait()

    def wait_store():
        pltpu.make_async_copy(obuf, out_hbm.at[0, pl.ds(0, PER_W)],
                              semS).wait()

    def compute(gbuf, row_base):
        @plsc.parallel_loop(0, CHUNK, 1, unroll=2)
        def _(c):
            for dblk in range(DBLK):
                o = pl.ds(dblk * 16, 16)
                m = _tree_max7([gbuf[K * c + k2, o] for k2 in range(K)])
                obuf[row_base + c, o] = m

    fire(gbufA, semA, 0, 0)
    fire(gbufB, semB, 1, 0)

    def batch_body(b, carry):
        @pl.when(b >= 1)
        def _():
            wait_store()

        def pair_body(j2, carry2):
            c0 = 2 * j2
            wait_gather(gbufA, semA)
            compute(gbufA, c0 * CHUNK)
            fire(gbufA, semA, c0 + 2, b)
            wait_gather(gbufB, semB)
            compute(gbufB, (c0 + 1) * CHUNK)
            fire(gbufB, semB, c0 + 3, b)
            return carry2

        lax.fori_loop(0, NCHUNK // 2 - 1, pair_body, 0)

        wait_gather(gbufA, semA)
        compute(gbufA, (NCHUNK - 2) * CHUNK)

        @pl.when(b < B - 1)
        def _():
            fire(gbufA, semA, 0, b + 1)

        wait_gather(gbufB, semB)
        compute(gbufB, (NCHUNK - 1) * CHUNK)

        @pl.when(b < B - 1)
        def _():
            fire(gbufB, semB, 1, b + 1)

        pltpu.async_copy(obuf, out_hbm.at[b, pl.ds(base_c, PER_W)], semS)
        return carry

    lax.fori_loop(0, B, batch_body, 0)
    wait_store()

    @pl.when(wid == NW - 1)
    def _():
        pltpu.sync_copy(idxt_hbm, idxt_raw)

        def tfire(b, carry):
            pltpu.async_copy(x_hbm.at[b].at[idxt_raw], gtail.at[b], semT)
            return carry

        lax.fori_loop(0, B, tfire, 0)

        def tdrain(b, carry):
            pltpu.make_async_copy(x_hbm.at[b].at[idxt_raw], gtail.at[b],
                                  semT).wait()
            for c in range(N_TAIL):
                for dblk in range(DBLK):
                    o = pl.ds(dblk * 16, 16)
                    m = _tree_max7([gtail[b, K * c + k2, o]
                                    for k2 in range(K)])
                    otail[c, o] = m
            pltpu.sync_copy(otail, out_hbm.at[b, pl.ds(N_MAIN, N_TAIL)])
            return carry

        lax.fori_loop(0, B, tdrain, 0)


def kernel(x, pool_idx):
    idx = pool_idx.astype(jnp.int32)
    idx_main = idx[:N_MAIN].reshape(N_MAIN * K)
    idx_tail = jnp.pad(idx[N_MAIN:].reshape(N_TAIL * K), (0, 16 - N_TAIL * K))

    mesh = plsc.VectorSubcoreMesh(core_axis_name="c", subcore_axis_name="s")
    f = pl.kernel(
        _pool_body,
        mesh=mesh,
        out_type=jax.ShapeDtypeStruct((B, N_COARSE, D), jnp.float32),
        scratch_types=[
            pltpu.VMEM((PER_W * K,), jnp.int32),
            pltpu.VMEM((IDX_C, D), jnp.float32),
            pltpu.VMEM((IDX_C, D), jnp.float32),
            pltpu.VMEM((PER_W, D), jnp.float32),
            pltpu.VMEM((16,), jnp.int32),
            pltpu.VMEM((B, 16, D), jnp.float32),
            pltpu.VMEM((N_TAIL, D), jnp.float32),
            pltpu.SemaphoreType.DMA,
            pltpu.SemaphoreType.DMA,
            pltpu.SemaphoreType.DMA,
            pltpu.SemaphoreType.DMA,
        ],
    )
    return f(x, idx_main, idx_tail)


# speedup vs baseline: 2.3898x; 1.1636x over previous
import jax
import jax.numpy as jnp
from jax import lax
from jax.experimental import pallas as pl
from jax.experimental.pallas import tpu as pltpu
from jax.experimental.pallas import tpu_sc as plsc

B = 8
N_FINE = 40962
D = 128
K = 7
N_COARSE = 10242

NW = 32
PER_W = 320
N_MAIN = NW * PER_W
N_TAIL = N_COARSE - N_MAIN
CHUNK = 16
NCHUNK = PER_W // CHUNK
IDX_C = CHUNK * K
DBLK = D // 16


def _tree_max7(rows):
    t0 = jnp.maximum(rows[0], rows[1])
    t1 = jnp.maximum(rows[2], rows[3])
    t2 = jnp.maximum(rows[4], rows[5])
    return jnp.maximum(jnp.maximum(t0, t1), jnp.maximum(t2, rows[6]))


NBUF = 4


def _pool_body(x_hbm, idxm_hbm, idxt_hbm, out_hbm,
               idx_v, gbuf0, gbuf1, gbuf2, gbuf3, obuf,
               idxt_raw, gtail, otail,
               sem0, sem1, sem2, sem3, semS, semT):
    gbufs = (gbuf0, gbuf1, gbuf2, gbuf3)
    sems = (sem0, sem1, sem2, sem3)
    wid = lax.axis_index("s") * 2 + lax.axis_index("c")
    base_c = wid * PER_W
    pltpu.sync_copy(idxm_hbm.at[pl.ds(wid * (PER_W * K), PER_W * K)], idx_v)

    def fire(gbuf, sem, j, b):
        idx_slice = idx_v.at[pl.ds(j * IDX_C, IDX_C)]
        pltpu.async_copy(x_hbm.at[b].at[idx_slice], gbuf, sem)

    def wait_gather(gbuf, sem):
        idx_slice = idx_v.at[pl.ds(0, IDX_C)]
        pltpu.make_async_copy(x_hbm.at[0].at[idx_slice], gbuf, sem).wait()

    def wait_store():
        pltpu.make_async_copy(obuf, out_hbm.at[0, pl.ds(0, PER_W)],
                              semS).wait()

    def compute(gbuf, row_base):
        @plsc.parallel_loop(0, CHUNK, 1, unroll=2)
        def _(c):
            for dblk in range(DBLK):
                o = pl.ds(dblk * 16, 16)
                m = _tree_max7([gbuf[K * c + k2, o] for k2 in range(K)])
                obuf[row_base + c, o] = m

    for p in range(NBUF):
        fire(gbufs[p], sems[p], p, 0)

    def batch_body(b, carry):
        @pl.when(b >= 1)
        def _():
            wait_store()

        def quad_body(j4, carry2):
            c0 = NBUF * j4
            for p in range(NBUF):
                wait_gather(gbufs[p], sems[p])
                compute(gbufs[p], (c0 + p) * CHUNK)
                fire(gbufs[p], sems[p], c0 + p + NBUF, b)
            return carry2

        lax.fori_loop(0, NCHUNK // NBUF - 1, quad_body, 0)

        for p in range(NBUF):
            wait_gather(gbufs[p], sems[p])
            compute(gbufs[p], (NCHUNK - NBUF + p) * CHUNK)

            @pl.when(b < B - 1)
            def _():
                fire(gbufs[p], sems[p], p, b + 1)

        pltpu.async_copy(obuf, out_hbm.at[b, pl.ds(base_c, PER_W)], semS)
        return carry

    lax.fori_loop(0, B, batch_body, 0)
    wait_store()

    @pl.when(wid == NW - 1)
    def _():
        pltpu.sync_copy(idxt_hbm, idxt_raw)

        def tfire(b, carry):
            pltpu.async_copy(x_hbm.at[b].at[idxt_raw], gtail.at[b], semT)
            return carry

        lax.fori_loop(0, B, tfire, 0)

        def tdrain(b, carry):
            pltpu.make_async_copy(x_hbm.at[b].at[idxt_raw], gtail.at[b],
                                  semT).wait()
            for c in range(N_TAIL):
                for dblk in range(DBLK):
                    o = pl.ds(dblk * 16, 16)
                    m = _tree_max7([gtail[b, K * c + k2, o]
                                    for k2 in range(K)])
                    otail[c, o] = m
            pltpu.sync_copy(otail, out_hbm.at[b, pl.ds(N_MAIN, N_TAIL)])
            return carry

        lax.fori_loop(0, B, tdrain, 0)


def kernel(x, pool_idx):
    idx = pool_idx.astype(jnp.int32)
    idx_main = idx[:N_MAIN].reshape(N_MAIN * K)
    idx_tail = jnp.pad(idx[N_MAIN:].reshape(N_TAIL * K), (0, 16 - N_TAIL * K))

    mesh = plsc.VectorSubcoreMesh(core_axis_name="c", subcore_axis_name="s")
    f = pl.kernel(
        _pool_body,
        mesh=mesh,
        out_type=jax.ShapeDtypeStruct((B, N_COARSE, D), jnp.float32),
        scratch_types=[
            pltpu.VMEM((PER_W * K,), jnp.int32),
            pltpu.VMEM((IDX_C, D), jnp.float32),
            pltpu.VMEM((IDX_C, D), jnp.float32),
            pltpu.VMEM((IDX_C, D), jnp.float32),
            pltpu.VMEM((IDX_C, D), jnp.float32),
            pltpu.VMEM((PER_W, D), jnp.float32),
            pltpu.VMEM((16,), jnp.int32),
            pltpu.VMEM((B, 16, D), jnp.float32),
            pltpu.VMEM((N_TAIL, D), jnp.float32),
            pltpu.SemaphoreType.DMA,
            pltpu.SemaphoreType.DMA,
            pltpu.SemaphoreType.DMA,
            pltpu.SemaphoreType.DMA,
            pltpu.SemaphoreType.DMA,
            pltpu.SemaphoreType.DMA,
        ],
    )
    return f(x, idx_main, idx_tail)
